# baseline (speedup 1.0000x reference)
import functools

import jax
import jax.numpy as jnp
from jax import lax
from jax.experimental import pallas as pl
from jax.experimental.pallas import tpu as pltpu
from jax.experimental.pallas import tpu_sc as plsc

NUM_NODES = 1000000
EMB = 64
N_IDX = 819200

NC, NS = 2, 16
NW = NC * NS

KSPLIT = 4
SLOT = 128
ROWS_PER_K = N_IDX // KSPLIT
ROWS_PER_W = ROWS_PER_K // NW
NSLOTS = ROWS_PER_W // SLOT
NGROUPS = N_IDX // SLOT
RING = 5
G = 3
NT = NSLOTS // RING

_mesh = plsc.VectorSubcoreMesh(core_axis_name="c", subcore_axis_name="s")
_linear = pltpu.CompilerParams(use_tc_tiling_on_sc=False)


_tiled = pltpu.CompilerParams(use_tc_tiling_on_sc=True)
IDXPAD = 8
STAGE = 64


def _make_gather(k):
    gbase = k * (NGROUPS // KSPLIT)

    @functools.partial(
        pl.kernel,
        out_type=jax.ShapeDtypeStruct((ROWS_PER_K, 128), jnp.float32),
        mesh=_mesh,
        scratch_types=[
            pltpu.VMEM((STAGE, SLOT), jnp.int32),
            pltpu.VMEM((RING, SLOT, 128), jnp.float32),
            pltpu.SemaphoreType.DMA((RING,)),
            pltpu.SemaphoreType.DMA((RING,)),
        ],
        compiler_params=_tiled,
        name=f"gather_rows{k}",
    )
    def _gather_sc(idx_hbm, tpad_hbm, out_hbm, idx_v, rows_v, gsem, ssem):
        wid = lax.axis_index("s") * NC + lax.axis_index("c")
        row0 = wid * ROWS_PER_W

        grow = gbase + wid * NSLOTS
        goff = lax.rem(grow, IDXPAD)
        gstart = pl.multiple_of(grow - goff, IDXPAD)
        pltpu.sync_copy(
            idx_hbm.at[pl.ds(gstart, STAGE)], idx_v
        )

        def fire_gather(s, b):
            pltpu.async_copy(
                tpad_hbm.at[idx_v.at[goff + s]], rows_v.at[b], gsem.at[b]
            )

        def wait_gather(s, b):
            pltpu.make_async_copy(
                tpad_hbm.at[idx_v.at[goff + s]], rows_v.at[b], gsem.at[b]
            ).wait()

        def fire_store(s, b):
            pltpu.async_copy(
                rows_v.at[b],
                out_hbm.at[pl.ds(row0 + s * SLOT, SLOT)],
                ssem.at[b],
            )

        def wait_store(s, b):
            pltpu.make_async_copy(
                rows_v.at[b],
                out_hbm.at[pl.ds(row0 + s * SLOT, SLOT)],
                ssem.at[b],
            ).wait()

        def body(t, carry):
            for b in range(RING):
                s = t * RING + b

                @pl.when(t > 0)
                def _():
                    wait_store(s - RING, b)

                fire_gather(s, b)

                bl = (b - G) % RING

                @pl.when(s >= G)
                def _():
                    wait_gather(s - G, bl)
                    fire_store(s - G, bl)

            return carry

        lax.fori_loop(0, NT, body, 0)

        for j in range(G):
            s = NSLOTS - G + j
            wait_gather(s, s % RING)
            fire_store(s, s % RING)
        for j in range(RING):
            s = NSLOTS - RING + j
            wait_store(s, s % RING)

    return _gather_sc


_gathers = [_make_gather(k) for k in range(KSPLIT)]


BLKT = 2048
NBLKT = -(-NUM_NODES // BLKT)


def _tpose_body(i_ref, o_ref):
    o_ref[:, :EMB] = i_ref[...].T


_tpose = pl.pallas_call(
    _tpose_body,
    grid=(NBLKT,),
    in_specs=[pl.BlockSpec((EMB, BLKT), lambda i: (0, i))],
    out_specs=pl.BlockSpec((BLKT, 128), lambda i: (i, 0)),
    out_shape=jax.ShapeDtypeStruct((NUM_NODES, 128), jnp.float32),
)


BLKC = 2048
NBLK = ROWS_PER_K // BLKC


def _depack_body(i_ref, o_ref):
    o_ref[...] = i_ref[:, :EMB].T


def _depack_body_alias(i_ref, prev_ref, o_ref):
    del prev_ref
    o_ref[...] = i_ref[:, :EMB].T


def _make_depack(k):
    common = dict(
        grid=(NBLK,),
        out_shape=jax.ShapeDtypeStruct((EMB, N_IDX), jnp.float32),
    )
    in_spec = pl.BlockSpec((BLKC, 128), lambda i: (i, 0))
    out_spec = pl.BlockSpec((EMB, BLKC), lambda i, _k=k: (0, _k * NBLK + i))
    if k == 0:
        return pl.pallas_call(
            _depack_body, in_specs=[in_spec], out_specs=out_spec, **common
        )
    return pl.pallas_call(
        _depack_body_alias,
        in_specs=[in_spec, pl.BlockSpec(memory_space=pl.ANY)],
        out_specs=out_spec,
        input_output_aliases={1: 0},
        **common,
    )


_depacks = [_make_depack(k) for k in range(KSPLIT)]


def kernel(node_idx, node_embs):
    tpad = jnp.pad(node_embs, ((0, 0), (0, 128 - EMB)))
    idx2d = jnp.pad(node_idx.reshape(NGROUPS, SLOT), ((0, IDXPAD), (0, 0)))
    out_t = None
    for k in range(KSPLIT):
        part = _gathers[k](idx2d, tpad)
        if k == 0:
            out_t = _depacks[0](part)
        else:
            out_t = _depacks[k](part, out_t)
    return out_t.T


# speedup vs baseline: 1.4868x; 1.4868x over previous
import functools

import jax
import jax.numpy as jnp
from jax import lax
from jax.experimental import pallas as pl
from jax.experimental.pallas import tpu as pltpu
from jax.experimental.pallas import tpu_sc as plsc

NUM_NODES = 1000000
EMB = 64
N_IDX = 819200

NC, NS = 2, 16
NW = NC * NS

KSPLIT = 8
SLOT = 128
ROWS_PER_K = N_IDX // KSPLIT
ROWS_PER_W = ROWS_PER_K // NW
NSLOTS = ROWS_PER_W
NSLOTS = ROWS_PER_W // SLOT
NGROUPS = N_IDX // SLOT
RING = 5
G = 3
NT = NSLOTS // RING

_mesh = plsc.VectorSubcoreMesh(core_axis_name="c", subcore_axis_name="s")
_linear = pltpu.CompilerParams(use_tc_tiling_on_sc=False)


def _make_gather(k):
    gbase = k * (NGROUPS // KSPLIT)

    @functools.partial(
        pl.kernel,
        out_type=jax.ShapeDtypeStruct((ROWS_PER_K, 128), jnp.float32),
        mesh=_mesh,
        scratch_types=[
            pltpu.VMEM((NSLOTS, SLOT), jnp.int32),
            pltpu.VMEM((RING, SLOT, EMB), jnp.float32),
            pltpu.SemaphoreType.DMA((RING,)),
            pltpu.SemaphoreType.DMA((RING,)),
        ],
        compiler_params=_linear,
        name=f"gather_rows{k}",
    )
    def _gather_sc(idx_hbm, tlin_hbm, out_hbm, idx_v, rows_v, gsem, ssem):
        wid = lax.axis_index("s") * NC + lax.axis_index("c")
        row0 = wid * ROWS_PER_W

        pltpu.sync_copy(idx_hbm.at[pl.ds(gbase + wid * NSLOTS, NSLOTS)], idx_v)

        def fire_gather(s, b):
            pltpu.async_copy(tlin_hbm.at[idx_v.at[s]], rows_v.at[b], gsem.at[b])

        def wait_gather(s, b):
            pltpu.make_async_copy(
                tlin_hbm.at[idx_v.at[s]], rows_v.at[b], gsem.at[b]
            ).wait()

        def fire_store(s, b):
            pltpu.async_copy(
                rows_v.at[b],
                out_hbm.at[pl.ds(row0 + s * SLOT, SLOT), pl.ds(0, EMB)],
                ssem.at[b],
            )

        def wait_store(s, b):
            pltpu.make_async_copy(
                rows_v.at[b],
                out_hbm.at[pl.ds(row0 + s * SLOT, SLOT), pl.ds(0, EMB)],
                ssem.at[b],
            ).wait()

        def body(t, carry):
            for b in range(RING):
                s = t * RING + b

                @pl.when(t > 0)
                def _():
                    wait_store(s - RING, b)

                fire_gather(s, b)

                bl = (b - G) % RING

                @pl.when(s >= G)
                def _():
                    wait_gather(s - G, bl)
                    fire_store(s - G, bl)

            return carry

        lax.fori_loop(0, NT, body, 0)

        for j in range(G):
            s = NSLOTS - G + j
            wait_gather(s, s % RING)
            fire_store(s, s % RING)
        for j in range(RING):
            s = NSLOTS - RING + j
            wait_store(s, s % RING)

    return _gather_sc


_gathers = [_make_gather(k) for k in range(KSPLIT)]


BLKT = 8192
NBLKT = -(-NUM_NODES // BLKT)


def _tpose_body(i_ref, o_ref):
    o_ref[:, :EMB] = i_ref[...].T


_tpose = pl.pallas_call(
    _tpose_body,
    grid=(NBLKT,),
    in_specs=[pl.BlockSpec((EMB, BLKT), lambda i: (0, i))],
    out_specs=pl.BlockSpec((BLKT, 128), lambda i: (i, 0)),
    out_shape=jax.ShapeDtypeStruct((NUM_NODES, 128), jnp.float32),
)


BLKC = 2048
NBLK = ROWS_PER_K // BLKC


def _depack_body(i_ref, o_ref):
    o_ref[...] = i_ref[:, :EMB].T


def _depack_body_alias(i_ref, prev_ref, o_ref):
    del prev_ref
    o_ref[...] = i_ref[:, :EMB].T


def _make_depack(k):
    common = dict(
        grid=(NBLK,),
        out_shape=jax.ShapeDtypeStruct((EMB, N_IDX), jnp.float32),
    )
    in_spec = pl.BlockSpec((BLKC, 128), lambda i: (i, 0))
    out_spec = pl.BlockSpec((EMB, BLKC), lambda i, _k=k: (0, _k * NBLK + i))
    if k == 0:
        return pl.pallas_call(
            _depack_body, in_specs=[in_spec], out_specs=out_spec, **common
        )
    return pl.pallas_call(
        _depack_body_alias,
        in_specs=[in_spec, pl.BlockSpec(memory_space=pl.ANY)],
        out_specs=out_spec,
        input_output_aliases={1: 0},
        **common,
    )


_depacks = [_make_depack(k) for k in range(KSPLIT)]


def kernel(node_idx, node_embs):
    tpad = _tpose(node_embs.T)
    tview = tpad.reshape(2 * NUM_NODES, EMB)
    idx2d = (node_idx * 2).reshape(NGROUPS, SLOT)
    out_t = None
    for k in range(KSPLIT):
        part = _gathers[k](idx2d, tview)
        if k == 0:
            out_t = _depacks[0](part)
        else:
            out_t = _depacks[k](part, out_t)
    return out_t.T


# speedup vs baseline: 1.7374x; 1.1685x over previous
import functools

import jax
import jax.numpy as jnp
from jax import lax
from jax.experimental import pallas as pl
from jax.experimental.pallas import tpu as pltpu
from jax.experimental.pallas import tpu_sc as plsc

NUM_NODES = 1000000
EMB = 64
N_IDX = 819200

NC, NS = 2, 16
NW = NC * NS

KSPLIT = 8
SLOT = 128
ROWS_PER_K = N_IDX // KSPLIT
ROWS_PER_W = ROWS_PER_K // NW
NSLOTS = ROWS_PER_W
NSLOTS = ROWS_PER_W // SLOT
NGROUPS = N_IDX // SLOT
RING = 5
G = 3
NT = NSLOTS // RING

_mesh = plsc.VectorSubcoreMesh(core_axis_name="c", subcore_axis_name="s")
_linear = pltpu.CompilerParams(use_tc_tiling_on_sc=False)


def _make_gather(k):
    gbase = k * (NGROUPS // KSPLIT)

    @functools.partial(
        pl.kernel,
        out_type=jax.ShapeDtypeStruct((ROWS_PER_K, 128), jnp.float32),
        mesh=_mesh,
        scratch_types=[
            pltpu.VMEM((NSLOTS, SLOT), jnp.int32),
            pltpu.VMEM((RING, SLOT, EMB), jnp.float32),
            pltpu.SemaphoreType.DMA((RING,)),
            pltpu.SemaphoreType.DMA((RING,)),
        ],
        compiler_params=_linear,
        name=f"gather_rows{k}",
    )
    def _gather_sc(idx_hbm, tlin_hbm, out_hbm, idx_v, rows_v, gsem, ssem):
        wid = lax.axis_index("s") * NC + lax.axis_index("c")
        row0 = wid * ROWS_PER_W

        pltpu.sync_copy(idx_hbm.at[pl.ds(gbase + wid * NSLOTS, NSLOTS)], idx_v)

        def fire_gather(s, b):
            pltpu.async_copy(tlin_hbm.at[idx_v.at[s]], rows_v.at[b], gsem.at[b])

        def wait_gather(s, b):
            pltpu.make_async_copy(
                tlin_hbm.at[idx_v.at[s]], rows_v.at[b], gsem.at[b]
            ).wait()

        def fire_store(s, b):
            pltpu.async_copy(
                rows_v.at[b],
                out_hbm.at[pl.ds(row0 + s * SLOT, SLOT), pl.ds(0, EMB)],
                ssem.at[b],
            )

        def wait_store(s, b):
            pltpu.make_async_copy(
                rows_v.at[b],
                out_hbm.at[pl.ds(row0 + s * SLOT, SLOT), pl.ds(0, EMB)],
                ssem.at[b],
            ).wait()

        def body(t, carry):
            for b in range(RING):
                s = t * RING + b

                @pl.when(t > 0)
                def _():
                    wait_store(s - RING, b)

                fire_gather(s, b)

                bl = (b - G) % RING

                @pl.when(s >= G)
                def _():
                    wait_gather(s - G, bl)
                    fire_store(s - G, bl)

            return carry

        lax.fori_loop(0, NT, body, 0)

        for j in range(G):
            s = NSLOTS - G + j
            wait_gather(s, s % RING)
            fire_store(s, s % RING)
        for j in range(RING):
            s = NSLOTS - RING + j
            wait_store(s, s % RING)

    return _gather_sc


_gathers = [_make_gather(k) for k in range(KSPLIT)]


BLKT = 16384
NBLKT = -(-NUM_NODES // BLKT)


def _tpose_body(i_ref, o_ref):
    o_ref[:, :EMB] = i_ref[...].T


_tpose = pl.pallas_call(
    _tpose_body,
    grid=(NBLKT,),
    in_specs=[pl.BlockSpec((EMB, BLKT), lambda i: (0, i))],
    out_specs=pl.BlockSpec((BLKT, 128), lambda i: (i, 0)),
    out_shape=jax.ShapeDtypeStruct((NUM_NODES, 128), jnp.float32),
)


BLKC = 4096
NBLK = ROWS_PER_K // BLKC


def _depack_body(i_ref, o_ref):
    o_ref[...] = i_ref[:, :EMB].T


def _depack_body_alias(i_ref, prev_ref, o_ref):
    del prev_ref
    o_ref[...] = i_ref[:, :EMB].T


def _make_depack(k):
    common = dict(
        grid=(NBLK,),
        out_shape=jax.ShapeDtypeStruct((EMB, N_IDX), jnp.float32),
    )
    in_spec = pl.BlockSpec((BLKC, 128), lambda i: (i, 0))
    out_spec = pl.BlockSpec((EMB, BLKC), lambda i, _k=k: (0, _k * NBLK + i))
    if k == 0:
        return pl.pallas_call(
            _depack_body, in_specs=[in_spec], out_specs=out_spec, **common
        )
    return pl.pallas_call(
        _depack_body_alias,
        in_specs=[in_spec, pl.BlockSpec(memory_space=pl.ANY)],
        out_specs=out_spec,
        input_output_aliases={1: 0},
        **common,
    )


_depacks = [_make_depack(k) for k in range(KSPLIT)]


def kernel(node_idx, node_embs):
    tpad = _tpose(node_embs.T)
    tview = tpad.reshape(2 * NUM_NODES, EMB)
    idx2d = (node_idx * 2).reshape(NGROUPS, SLOT)
    out_t = None
    for k in range(KSPLIT):
        part = _gathers[k](idx2d, tview)
        if k == 0:
            out_t = _depacks[0](part)
        else:
            out_t = _depacks[k](part, out_t)
    return out_t.T
